# baseline (speedup 1.0000x reference)
import functools

import jax
import jax.numpy as jnp
from jax import lax
from jax.experimental import pallas as pl
from jax.experimental.pallas import tpu as pltpu, tpu_sc as plsc

HID = 64
BATCH = 16384
HIST = 200
NROW = 5000

NC = 2
NS = 16
NW = NC * NS

HT = 8
NQ = NW // HT
TQ = HIST // NQ
NBT = BATCH // 128
BT = 16
NST = NBT // BT


def _embed_sc(xT, tableT):
    mesh = plsc.VectorSubcoreMesh(core_axis_name="c", subcore_axis_name="s")

    @functools.partial(
        pl.kernel,
        out_type=jax.ShapeDtypeStruct((HIST, HT, NBT, 8, 128), jnp.float32),
        mesh=mesh,
        scratch_types=[pltpu.VMEM((NROW,), jnp.float32)] * 8
        + [
            pltpu.VMEM((2, BATCH), jnp.int32),
            pltpu.VMEM((2, BT, 8, 128), jnp.float32),
            pltpu.SemaphoreType.DMA,
            pltpu.SemaphoreType.DMA,
            pltpu.SemaphoreType.DMA,
            pltpu.SemaphoreType.DMA,
            pltpu.SemaphoreType.DMA,
        ],
        compiler_params=pltpu.CompilerParams(
            use_tc_tiling_on_sc=False, needs_layout_passes=False
        ),
    )
    def k(xT_hbm, tT_hbm, out_hbm, *refs):
        tab = refs[:8]
        idx_v, out_v, tsem, i0, i1, o0, o1 = refs[8:]
        isem = (i0, i1)
        osem = (o0, o1)
        wid = lax.axis_index("s") * NC + lax.axis_index("c")
        ht = wid % HT
        tbase = (wid // HT) * TQ

        for h in range(8):
            pltpu.async_copy(tT_hbm.at[ht * 8 + h], tab[h], tsem)
        for p in range(2):
            pltpu.async_copy(xT_hbm.at[tbase + p], idx_v.at[p], isem[p])
        for h in range(8):
            pltpu.make_async_copy(tT_hbm.at[0], tab[h], tsem).wait()

        def drain_group(q):
            pltpu.make_async_copy(
                out_v.at[q], out_hbm.at[0, 0, pl.ds(0, BT)], osem[q]
            ).wait()

        @pl.loop(0, TQ, step=2)
        def _trow(tt):
            for p in range(2):
                t = tbase + tt + p
                pltpu.make_async_copy(
                    xT_hbm.at[0], idx_v.at[p], isem[p]
                ).wait()

                @pl.loop(0, NST, step=2)
                def _group(st):
                    for q in range(2):
                        bt0 = (st + q) * BT
                        if p == 0:
                            @pl.when((tt > 0) | (st > 0))
                            def _():
                                drain_group(q)
                        else:
                            drain_group(q)

                        @plsc.parallel_loop(0, BT * 8, unroll=16)
                        def _gather(i):
                            jj = i // 8
                            g = i % 8
                            vidx = idx_v[
                                p, pl.ds(bt0 * 128 + jj * 128 + g * 16, 16)
                            ]
                            for h in range(8):
                                out_v[q, jj, h, pl.ds(g * 16, 16)] = (
                                    plsc.load_gather(tab[h], [vidx])
                                )

                        pltpu.async_copy(
                            out_v.at[q],
                            out_hbm.at[t, ht, pl.ds(bt0, BT)],
                            osem[q],
                        )

                @pl.when(t + 2 < tbase + TQ)
                def _():
                    pltpu.async_copy(
                        xT_hbm.at[t + 2], idx_v.at[p], isem[p]
                    )

        for q in range(2):
            drain_group(q)

    return k(xT, tableT)


def kernel(x, table):
    out5 = _embed_sc(x.T, table.T)
    return out5.transpose((2, 4, 0, 1, 3)).reshape(BATCH, HIST, HID)


# speedup vs baseline: 1.0273x; 1.0273x over previous
import functools

import jax
import jax.numpy as jnp
from jax import lax
from jax.experimental import pallas as pl
from jax.experimental.pallas import tpu as pltpu, tpu_sc as plsc

HID = 64
BATCH = 16384
HIST = 200
NROW = 5000

NC = 2
NS = 16
NW = NC * NS

HT = 8
NQ = NW // HT
TQ = HIST // NQ
NBT = BATCH // 128
BT = 16
NST = NBT // BT


def _embed_sc(xP, tableT):
    mesh = plsc.VectorSubcoreMesh(core_axis_name="c", subcore_axis_name="s")

    @functools.partial(
        pl.kernel,
        out_type=jax.ShapeDtypeStruct((HIST, HT, NBT, 8, 128), jnp.float32),
        mesh=mesh,
        scratch_types=[pltpu.VMEM((NROW,), jnp.float32)] * 8
        + [
            pltpu.VMEM((2, NBT, 128), jnp.int32),
            pltpu.VMEM((2, BT, 8, 128), jnp.float32),
            pltpu.SemaphoreType.DMA,
            pltpu.SemaphoreType.DMA,
            pltpu.SemaphoreType.DMA,
            pltpu.SemaphoreType.DMA,
            pltpu.SemaphoreType.DMA,
        ],
        compiler_params=pltpu.CompilerParams(
            use_tc_tiling_on_sc=False, needs_layout_passes=False
        ),
    )
    def k(xP_hbm, tT_hbm, out_hbm, *refs):
        tab = refs[:8]
        idx_v, out_v, tsem, i0, i1, o0, o1 = refs[8:]
        isem = (i0, i1)
        osem = (o0, o1)
        wid = lax.axis_index("s") * NC + lax.axis_index("c")
        ht = wid % HT
        tbase = (wid // HT) * TQ

        for h in range(8):
            pltpu.async_copy(tT_hbm.at[ht * 8 + h], tab[h], tsem)
        def load_idx_row(t, p):
            pltpu.async_copy(
                xP_hbm.at[t // 8, pl.ds(0, NBT), t % 8], idx_v.at[p], isem[p]
            )

        for p in range(2):
            load_idx_row(tbase + p, p)
        for h in range(8):
            pltpu.make_async_copy(tT_hbm.at[0], tab[h], tsem).wait()

        def drain_group(q):
            pltpu.make_async_copy(
                out_v.at[q], out_hbm.at[0, 0, pl.ds(0, BT)], osem[q]
            ).wait()

        @pl.loop(0, TQ, step=2)
        def _trow(tt):
            for p in range(2):
                t = tbase + tt + p
                pltpu.make_async_copy(
                    xP_hbm.at[0, pl.ds(0, NBT), 0], idx_v.at[p], isem[p]
                ).wait()

                @pl.loop(0, NST, step=2)
                def _group(st):
                    for q in range(2):
                        bt0 = (st + q) * BT
                        if p == 0:
                            @pl.when((tt > 0) | (st > 0))
                            def _():
                                drain_group(q)
                        else:
                            drain_group(q)

                        @plsc.parallel_loop(0, BT * 8, unroll=16)
                        def _gather(i):
                            jj = i // 8
                            g = i % 8
                            vidx = idx_v[p, bt0 + jj, pl.ds(g * 16, 16)]
                            for h in range(8):
                                out_v[q, jj, h, pl.ds(g * 16, 16)] = (
                                    plsc.load_gather(tab[h], [vidx])
                                )

                        pltpu.async_copy(
                            out_v.at[q],
                            out_hbm.at[t, ht, pl.ds(bt0, BT)],
                            osem[q],
                        )

                @pl.when(t + 2 < tbase + TQ)
                def _():
                    load_idx_row(t + 2, p)

        for q in range(2):
            drain_group(q)

    return k(xP, tableT)


def kernel(x, table):
    xP = x.T.reshape(HIST // 8, 8, NBT, 128).transpose((0, 2, 1, 3))
    out5 = _embed_sc(xP, table.T)
    return out5.transpose((2, 4, 0, 1, 3)).reshape(BATCH, HIST, HID)
